# baseline (speedup 1.0000x reference)
import numpy as np
import jax
import jax.numpy as jnp
from jax import lax
from jax.experimental import pallas as pl
from jax.experimental.pallas import tpu as pltpu, tpu_sc as plsc

NEURONS_C = 32768
K_C = 64
BATCH_C = 128
NWORKERS = 32
ROWS_PER_WORKER = BATCH_C // NWORKERS
NV_ROW = NEURONS_C // 16
NSEG = NV_ROW // 16

MIN32 = np.int32(-2**31)
M7F = np.int32(0x7FFFFFFF)
NEG_INF = np.float32(-np.inf)


def _keyf(u):
    sk = u ^ MIN32
    return lax.bitcast_convert_type(
        sk ^ (lax.shift_right_arithmetic(sk, 31) & M7F), jnp.float32)


def _sc_body(s_hbm, o_hbm, rowa_v, rowb_v, gmax_v, cmax_v, glist_v, cv_v,
             in0_sem, in1_sem, out0_sem, out1_sem):
    wid = lax.axis_index("s") * 2 + lax.axis_index("c")
    lane = lax.iota(jnp.int32, 16)
    lane16 = lane * 16
    zeros16 = jnp.zeros((16,), jnp.int32)
    ninf16 = jnp.full((16,), NEG_INF)

    def compute_row(buf, row, mid_cb=None):
        @plsc.parallel_loop(0, NSEG, unroll=4)
        def _g(t):
            vs = [buf[pl.ds(t * 256 + k * 16, 16)] for k in range(16)]
            while len(vs) > 1:
                vs = [jnp.maximum(vs[2 * a], vs[2 * a + 1])
                      for a in range(len(vs) // 2)]
            gmax_v[pl.ds(t * 16, 16)] = vs[0]

        for t in range(16):
            vs = [gmax_v[pl.ds(t * 128 + k * 16, 16)] for k in range(8)]
            while len(vs) > 1:
                vs = [jnp.maximum(vs[2 * a], vs[2 * a + 1])
                      for a in range(len(vs) // 2)]
            cmax_v[pl.ds(t * 16, 16)] = vs[0]

        if mid_cb is not None:
            mid_cb()

        u = zeros16
        for b in range(31, -1, -1):
            bit = MIN32 if b == 31 else np.int32(1 << b)
            t_f = _keyf(u | bit)
            cnt = zeros16
            for t in range(16):
                m = cmax_v[pl.ds(t * 16, 16)] >= t_f
                cnt = cnt + plsc.all_reduce_population_count(m)
            u = jnp.where(cnt >= K_C, u | bit, u)
        uc = u
        c64_f = _keyf(u)

        @plsc.parallel_loop(0, NSEG, unroll=4, carry=zeros16)
        def ng(t, c_c):
            m = gmax_v[pl.ds(t * 16, 16)] >= c64_f
            pos = c_c + plsc.cumsum(m.astype(jnp.int32)) - 1
            plsc.store_scatter(glist_v, [pos], lane + t * 16, mask=m)
            return c_c + plsc.all_reduce_population_count(m)
        ng_s = ng[0]

        nblk = (ng_s + 15) >> 4

        @plsc.parallel_loop(0, nblk)
        def _cg(blk):
            gid = glist_v[pl.ds(blk * 16, 16)] & 2047
            base = lax.shift_left(lax.shift_right_arithmetic(gid, 4), 8) + (
                gid & 15)
            for k in range(16):
                cv_v[pl.ds((blk * 16 + k) * 16, 16)] = plsc.load_gather(
                    buf, [base + k * 16])

        valid = ng_s - (nblk - 1) * 16
        for k in range(16):
            o = ((nblk - 1) * 16 + k) * 16
            v = cv_v[pl.ds(o, 16)]
            cv_v[pl.ds(o, 16)] = jnp.where(lane < valid, v, ninf16)
        nv4 = nblk * 4

        def count_cmp(t_f, strict=False):
            @plsc.parallel_loop(0, nv4, unroll=2, carry=zeros16)
            def acc(t, acc_c):
                for q in range(4):
                    x = cv_v[pl.ds(t * 64 + q * 16, 16)]
                    m = (x > t_f) if strict else (x >= t_f)
                    acc_c = acc_c + plsc.all_reduce_population_count(m)
                return acc_c
            return acc

        mm = [cmax_v[pl.ds(k * 16, 16)] for k in range(16)]
        while len(mm) > 1:
            mm = [jnp.maximum(mm[2 * a], mm[2 * a + 1])
                  for a in range(len(mm) // 2)]
        hmax = jnp.max(mm[0]) + jnp.zeros((16,), jnp.float32)
        ih = lax.bitcast_convert_type(hmax, jnp.int32)
        uh = ih ^ (lax.shift_right_arithmetic(ih, 31) | MIN32)
        d = uc ^ uh
        df = d.astype(jnp.float32)
        hb = (lax.shift_right_logical(
            lax.bitcast_convert_type(df, jnp.int32), 23) & 255) - 127
        hb = jnp.where(d < 0, 31, hb)
        nb = jnp.maximum(hb + 1, 0)
        hbc = jnp.maximum(hb, 0)
        u0 = jnp.where(d == 0, uc, uc & ~(lax.shift_left(2, hbc) - 1))
        ones_i = jnp.ones((16,), jnp.int32)

        def _step(i, u):
            bit = lax.shift_left(ones_i, hb - i)
            cnt = count_cmp(_keyf(u | bit))
            return jnp.where(cnt >= K_C, u | bit, u)

        u = lax.fori_loop(0, nb[0], _step, u0)
        thr_f = _keyf(u)

        @plsc.parallel_loop(0, nv4, unroll=2, carry=(zeros16, zeros16))
        def gecnt(t, carry):
            g_c, e_c = carry
            for q in range(4):
                x = cv_v[pl.ds(t * 64 + q * 16, 16)]
                g_c = g_c + plsc.all_reduce_population_count(x > thr_f)
                e_c = e_c + plsc.all_reduce_population_count(x >= thr_f)
            return (g_c, e_c)
        n_gt, n_ge = gecnt
        need = K_C - n_gt
        n_eq = n_ge - n_gt

        def tie_search():
            jcut0 = zeros16
            for b in range(14, -1, -1):
                candj = jcut0 | np.int32(1 << b)

                @plsc.parallel_loop(0, NV_ROW, unroll=4, carry=zeros16)
                def cj(i, c_c):
                    x = buf[pl.ds(i * 16, 16)]
                    m = (x == thr_f) & (lane + i * 16 >= candj)
                    return c_c + plsc.all_reduce_population_count(m)
                jcut0 = jnp.where(cj >= need, candj, jcut0)
            return jcut0

        tie_mode = ((n_eq != need).astype(jnp.int32))[0] != 0
        jcut = lax.cond(tie_mode, tie_search, lambda: zeros16)

        @pl.when(jnp.logical_not(tie_mode))
        def _mask_fast():
            @plsc.parallel_loop(0, NV_ROW, unroll=8)
            def _mk(i):
                x = buf[pl.ds(i * 16, 16)]
                buf[pl.ds(i * 16, 16)] = jnp.where(x >= thr_f, x, 0.0)

        @pl.when(tie_mode)
        def _mask_tie():
            @plsc.parallel_loop(0, NV_ROW, unroll=4)
            def _mk(i):
                x = buf[pl.ds(i * 16, 16)]
                keep = (x > thr_f) | ((x == thr_f) & (lane + i * 16 >= jcut))
                buf[pl.ds(i * 16, 16)] = jnp.where(keep, x, 0.0)

    base = wid * ROWS_PER_WORKER

    def in_dma(buf, row, sem):
        return pltpu.make_async_copy(s_hbm.at[row], buf, sem)

    def out_dma(buf, row, sem):
        return pltpu.make_async_copy(buf, o_hbm.at[row], sem)

    in_dma(rowa_v, base, in0_sem).start()

    def _pair(p, _):
        r0 = base + 2 * p
        in_dma(rowa_v, r0, in0_sem).wait()

        @pl.when(p > 0)
        def _w():
            out_dma(rowb_v, r0 - 1, out1_sem).wait()
        in_dma(rowb_v, r0 + 1, in1_sem).start()
        compute_row(rowa_v, r0)
        out_dma(rowa_v, r0, out0_sem).start()
        in_dma(rowb_v, r0 + 1, in1_sem).wait()

        def _recycle_a():
            out_dma(rowa_v, r0, out0_sem).wait()

            @pl.when(p == 0)
            def _n():
                in_dma(rowa_v, r0 + 2, in0_sem).start()

        compute_row(rowb_v, r0 + 1, mid_cb=_recycle_a)
        out_dma(rowb_v, r0 + 1, out1_sem).start()
        return 0

    lax.fori_loop(0, ROWS_PER_WORKER // 2, _pair, 0)
    out_dma(rowb_v, base + 3, out1_sem).wait()


@jax.jit
def kernel(s):
    mesh = plsc.VectorSubcoreMesh(core_axis_name="c", subcore_axis_name="s",
                                  num_cores=2, num_subcores=16)
    return pl.kernel(
        _sc_body,
        out_type=jax.ShapeDtypeStruct((BATCH_C, NEURONS_C), jnp.float32),
        mesh=mesh,
        compiler_params=pltpu.CompilerParams(needs_layout_passes=False),
        scratch_types=[
            pltpu.VMEM((NEURONS_C,), jnp.float32),
            pltpu.VMEM((NEURONS_C,), jnp.float32),
            pltpu.VMEM((NV_ROW,), jnp.float32),
            pltpu.VMEM((256,), jnp.float32),
            pltpu.VMEM((NV_ROW + 16,), jnp.int32),
            pltpu.VMEM((NEURONS_C + 64,), jnp.float32),
            pltpu.SemaphoreType.DMA,
            pltpu.SemaphoreType.DMA,
            pltpu.SemaphoreType.DMA,
            pltpu.SemaphoreType.DMA,
        ],
    )(s)


# speedup vs baseline: 1.0142x; 1.0142x over previous
import numpy as np
import jax
import jax.numpy as jnp
from jax import lax
from jax.experimental import pallas as pl
from jax.experimental.pallas import tpu as pltpu, tpu_sc as plsc

NEURONS_C = 32768
K_C = 64
BATCH_C = 128
NWORKERS = 32
ROWS_PER_WORKER = BATCH_C // NWORKERS
NV_ROW = NEURONS_C // 16
NSEG = NV_ROW // 16

MIN32 = np.int32(-2**31)
M7F = np.int32(0x7FFFFFFF)
NEG_INF = np.float32(-np.inf)


def _keyf(u):
    sk = u ^ MIN32
    return lax.bitcast_convert_type(
        sk ^ (lax.shift_right_arithmetic(sk, 31) & M7F), jnp.float32)


def _sc_body(s_hbm, o_hbm, rowa_v, rowb_v, gmax_v, cmax_v, glist_v, cv_v,
             in0_sem, in1_sem, out0_sem, out1_sem):
    wid = lax.axis_index("s") * 2 + lax.axis_index("c")
    lane = lax.iota(jnp.int32, 16)
    lane16 = lane * 16
    zeros16 = jnp.zeros((16,), jnp.int32)
    ninf16 = jnp.full((16,), NEG_INF)

    def compute_row(buf, row, mid_cb=None):
        @plsc.parallel_loop(0, NSEG, unroll=2)
        def _g(t):
            vs = [buf[pl.ds(t * 256 + k * 16, 16)] for k in range(16)]
            while len(vs) > 1:
                vs = [jnp.maximum(vs[2 * a], vs[2 * a + 1])
                      for a in range(len(vs) // 2)]
            gmax_v[pl.ds(t * 16, 16)] = vs[0]

        for t in range(16):
            vs = [gmax_v[pl.ds(t * 128 + k * 16, 16)] for k in range(8)]
            while len(vs) > 1:
                vs = [jnp.maximum(vs[2 * a], vs[2 * a + 1])
                      for a in range(len(vs) // 2)]
            cmax_v[pl.ds(t * 16, 16)] = vs[0]

        if mid_cb is not None:
            mid_cb()

        u = zeros16
        for b in range(31, -1, -1):
            bit = MIN32 if b == 31 else np.int32(1 << b)
            t_f = _keyf(u | bit)
            cnt = zeros16
            for t in range(16):
                m = cmax_v[pl.ds(t * 16, 16)] >= t_f
                cnt = cnt + plsc.all_reduce_population_count(m)
            u = jnp.where(cnt >= K_C, u | bit, u)
        uc = u
        c64_f = _keyf(u)

        @plsc.parallel_loop(0, NSEG, unroll=4, carry=zeros16)
        def ng(t, c_c):
            m = gmax_v[pl.ds(t * 16, 16)] >= c64_f
            pos = c_c + plsc.cumsum(m.astype(jnp.int32)) - 1
            plsc.store_scatter(glist_v, [pos], lane + t * 16, mask=m)
            return c_c + plsc.all_reduce_population_count(m)
        ng_s = ng[0]

        nblk = (ng_s + 15) >> 4

        @plsc.parallel_loop(0, nblk)
        def _cg(blk):
            gid = glist_v[pl.ds(blk * 16, 16)] & 2047
            base = lax.shift_left(lax.shift_right_arithmetic(gid, 4), 8) + (
                gid & 15)
            for k in range(16):
                cv_v[pl.ds((blk * 16 + k) * 16, 16)] = plsc.load_gather(
                    buf, [base + k * 16])

        valid = ng_s - (nblk - 1) * 16
        for k in range(16):
            o = ((nblk - 1) * 16 + k) * 16
            v = cv_v[pl.ds(o, 16)]
            cv_v[pl.ds(o, 16)] = jnp.where(lane < valid, v, ninf16)
        nv4 = nblk * 4

        def count_cmp(t_f, strict=False):
            @plsc.parallel_loop(0, nv4, unroll=2, carry=zeros16)
            def acc(t, acc_c):
                for q in range(4):
                    x = cv_v[pl.ds(t * 64 + q * 16, 16)]
                    m = (x > t_f) if strict else (x >= t_f)
                    acc_c = acc_c + plsc.all_reduce_population_count(m)
                return acc_c
            return acc

        mm = [cmax_v[pl.ds(k * 16, 16)] for k in range(16)]
        while len(mm) > 1:
            mm = [jnp.maximum(mm[2 * a], mm[2 * a + 1])
                  for a in range(len(mm) // 2)]
        hmax = jnp.max(mm[0]) + jnp.zeros((16,), jnp.float32)
        ih = lax.bitcast_convert_type(hmax, jnp.int32)
        uh = ih ^ (lax.shift_right_arithmetic(ih, 31) | MIN32)
        d = uc ^ uh
        df = d.astype(jnp.float32)
        hb = (lax.shift_right_logical(
            lax.bitcast_convert_type(df, jnp.int32), 23) & 255) - 127
        hb = jnp.where(d < 0, 31, hb)
        nb = jnp.maximum(hb + 1, 0)
        hbc = jnp.maximum(hb, 0)
        u0 = jnp.where(d == 0, uc, uc & ~(lax.shift_left(2, hbc) - 1))
        ones_i = jnp.ones((16,), jnp.int32)

        def _step(i, u):
            bit = lax.shift_left(ones_i, hb - i)
            cnt = count_cmp(_keyf(u | bit))
            return jnp.where(cnt >= K_C, u | bit, u)

        u = lax.fori_loop(0, nb[0], _step, u0)
        thr_f = _keyf(u)

        @plsc.parallel_loop(0, nv4, unroll=2, carry=(zeros16, zeros16))
        def gecnt(t, carry):
            g_c, e_c = carry
            for q in range(4):
                x = cv_v[pl.ds(t * 64 + q * 16, 16)]
                g_c = g_c + plsc.all_reduce_population_count(x > thr_f)
                e_c = e_c + plsc.all_reduce_population_count(x >= thr_f)
            return (g_c, e_c)
        n_gt, n_ge = gecnt
        need = K_C - n_gt
        n_eq = n_ge - n_gt

        def tie_search():
            jcut0 = zeros16
            for b in range(14, -1, -1):
                candj = jcut0 | np.int32(1 << b)

                @plsc.parallel_loop(0, NV_ROW, unroll=4, carry=zeros16)
                def cj(i, c_c):
                    x = buf[pl.ds(i * 16, 16)]
                    m = (x == thr_f) & (lane + i * 16 >= candj)
                    return c_c + plsc.all_reduce_population_count(m)
                jcut0 = jnp.where(cj >= need, candj, jcut0)
            return jcut0

        tie_mode = ((n_eq != need).astype(jnp.int32))[0] != 0
        jcut = lax.cond(tie_mode, tie_search, lambda: zeros16)

        @pl.when(jnp.logical_not(tie_mode))
        def _mask_fast():
            @plsc.parallel_loop(0, NV_ROW, unroll=8)
            def _mk(i):
                x = buf[pl.ds(i * 16, 16)]
                buf[pl.ds(i * 16, 16)] = jnp.where(x >= thr_f, x, 0.0)

        @pl.when(tie_mode)
        def _mask_tie():
            @plsc.parallel_loop(0, NV_ROW, unroll=4)
            def _mk(i):
                x = buf[pl.ds(i * 16, 16)]
                keep = (x > thr_f) | ((x == thr_f) & (lane + i * 16 >= jcut))
                buf[pl.ds(i * 16, 16)] = jnp.where(keep, x, 0.0)

    base = wid * ROWS_PER_WORKER

    def in_dma(buf, row, sem):
        return pltpu.make_async_copy(s_hbm.at[row], buf, sem)

    def out_dma(buf, row, sem):
        return pltpu.make_async_copy(buf, o_hbm.at[row], sem)

    in_dma(rowa_v, base, in0_sem).start()

    def _pair(p, _):
        r0 = base + 2 * p
        in_dma(rowa_v, r0, in0_sem).wait()

        @pl.when(p > 0)
        def _w():
            out_dma(rowb_v, r0 - 1, out1_sem).wait()
        in_dma(rowb_v, r0 + 1, in1_sem).start()
        compute_row(rowa_v, r0)
        out_dma(rowa_v, r0, out0_sem).start()
        in_dma(rowb_v, r0 + 1, in1_sem).wait()

        def _recycle_a():
            out_dma(rowa_v, r0, out0_sem).wait()

            @pl.when(p == 0)
            def _n():
                in_dma(rowa_v, r0 + 2, in0_sem).start()

        compute_row(rowb_v, r0 + 1, mid_cb=_recycle_a)
        out_dma(rowb_v, r0 + 1, out1_sem).start()
        return 0

    lax.fori_loop(0, ROWS_PER_WORKER // 2, _pair, 0)
    out_dma(rowb_v, base + 3, out1_sem).wait()


@jax.jit
def kernel(s):
    mesh = plsc.VectorSubcoreMesh(core_axis_name="c", subcore_axis_name="s",
                                  num_cores=2, num_subcores=16)
    return pl.kernel(
        _sc_body,
        out_type=jax.ShapeDtypeStruct((BATCH_C, NEURONS_C), jnp.float32),
        mesh=mesh,
        compiler_params=pltpu.CompilerParams(needs_layout_passes=False),
        scratch_types=[
            pltpu.VMEM((NEURONS_C,), jnp.float32),
            pltpu.VMEM((NEURONS_C,), jnp.float32),
            pltpu.VMEM((NV_ROW,), jnp.float32),
            pltpu.VMEM((256,), jnp.float32),
            pltpu.VMEM((NV_ROW + 16,), jnp.int32),
            pltpu.VMEM((NEURONS_C + 64,), jnp.float32),
            pltpu.SemaphoreType.DMA,
            pltpu.SemaphoreType.DMA,
            pltpu.SemaphoreType.DMA,
            pltpu.SemaphoreType.DMA,
        ],
    )(s)


# speedup vs baseline: 1.0451x; 1.0305x over previous
import numpy as np
import jax
import jax.numpy as jnp
from jax import lax
from jax.experimental import pallas as pl
from jax.experimental.pallas import tpu as pltpu, tpu_sc as plsc

NEURONS_C = 32768
K_C = 64
BATCH_C = 128
NWORKERS = 32
ROWS_PER_WORKER = BATCH_C // NWORKERS
NV_ROW = NEURONS_C // 16
NSEG = NV_ROW // 16

MIN32 = np.int32(-2**31)
M7F = np.int32(0x7FFFFFFF)
NEG_INF = np.float32(-np.inf)


def _keyf(u):
    sk = u ^ MIN32
    return lax.bitcast_convert_type(
        sk ^ (lax.shift_right_arithmetic(sk, 31) & M7F), jnp.float32)


def _sc_body(s_hbm, o_hbm, rowa_v, rowb_v, gmax_v, cmax_v, glist_v, cv_v,
             in0_sem, in1_sem, out0_sem, out1_sem):
    wid = lax.axis_index("s") * 2 + lax.axis_index("c")
    lane = lax.iota(jnp.int32, 16)
    lane16 = lane * 16
    zeros16 = jnp.zeros((16,), jnp.int32)
    ninf16 = jnp.full((16,), NEG_INF)

    def compute_row(buf, row, mid_cb=None):
        @plsc.parallel_loop(0, NSEG, unroll=2)
        def _g(t):
            vs = [buf[pl.ds(t * 256 + k * 16, 16)] for k in range(16)]
            while len(vs) > 1:
                vs = [jnp.maximum(vs[2 * a], vs[2 * a + 1])
                      for a in range(len(vs) // 2)]
            gmax_v[pl.ds(t * 16, 16)] = vs[0]

        for t in range(16):
            vs = [gmax_v[pl.ds(t * 128 + k * 16, 16)] for k in range(8)]
            while len(vs) > 1:
                vs = [jnp.maximum(vs[2 * a], vs[2 * a + 1])
                      for a in range(len(vs) // 2)]
            cmax_v[pl.ds(t * 16, 16)] = vs[0]

        if mid_cb is not None:
            mid_cb()

        u = zeros16
        for b in range(31, -1, -1):
            bit = MIN32 if b == 31 else np.int32(1 << b)
            t_f = _keyf(u | bit)
            cnt = zeros16
            for t in range(16):
                m = cmax_v[pl.ds(t * 16, 16)] >= t_f
                cnt = cnt + plsc.all_reduce_population_count(m)
            u = jnp.where(cnt >= K_C, u | bit, u)
        uc = u
        c64_f = _keyf(u)

        @plsc.parallel_loop(0, NSEG, unroll=4, carry=zeros16)
        def ng(t, c_c):
            m = gmax_v[pl.ds(t * 16, 16)] >= c64_f
            pos = c_c + plsc.cumsum(m.astype(jnp.int32)) - 1
            plsc.store_scatter(glist_v, [pos], lane + t * 16, mask=m)
            return c_c + plsc.all_reduce_population_count(m)
        ng_s = ng[0]

        @plsc.parallel_loop(0, ng_s, unroll=2)
        def _cg(g):
            gid = plsc.load_gather(glist_v, [g + zeros16])
            base = lax.shift_left(lax.shift_right_arithmetic(gid, 4), 8)
            idx = base + lane16 + (gid & 15)
            cv_v[pl.ds(g * 16, 16)] = plsc.load_gather(buf, [idx])

        for t in range(3):
            cv_v[pl.ds((ng_s + t) * 16, 16)] = ninf16
        nv4 = (ng_s + 3) >> 2

        def count_cmp(t_f, strict=False):
            @plsc.parallel_loop(0, nv4, unroll=2, carry=zeros16)
            def acc(t, acc_c):
                for q in range(4):
                    x = cv_v[pl.ds(t * 64 + q * 16, 16)]
                    m = (x > t_f) if strict else (x >= t_f)
                    acc_c = acc_c + plsc.all_reduce_population_count(m)
                return acc_c
            return acc

        mm = [cmax_v[pl.ds(k * 16, 16)] for k in range(16)]
        while len(mm) > 1:
            mm = [jnp.maximum(mm[2 * a], mm[2 * a + 1])
                  for a in range(len(mm) // 2)]
        hmax = jnp.max(mm[0]) + jnp.zeros((16,), jnp.float32)
        ih = lax.bitcast_convert_type(hmax, jnp.int32)
        uh = ih ^ (lax.shift_right_arithmetic(ih, 31) | MIN32)
        d = uc ^ uh
        df = d.astype(jnp.float32)
        hb = (lax.shift_right_logical(
            lax.bitcast_convert_type(df, jnp.int32), 23) & 255) - 127
        hb = jnp.where(d < 0, 31, hb)
        nb = jnp.maximum(hb + 1, 0)
        hbc = jnp.maximum(hb, 0)
        u0 = jnp.where(d == 0, uc, uc & ~(lax.shift_left(2, hbc) - 1))
        ones_i = jnp.ones((16,), jnp.int32)

        def _step(i, u):
            bit = lax.shift_left(ones_i, hb - i)
            cnt = count_cmp(_keyf(u | bit))
            return jnp.where(cnt >= K_C, u | bit, u)

        u = lax.fori_loop(0, nb[0], _step, u0)
        thr_f = _keyf(u)

        n_gt = count_cmp(thr_f, strict=True)
        need = K_C - n_gt
        n_eq = count_cmp(thr_f) - n_gt

        def tie_search():
            jcut0 = zeros16
            for b in range(14, -1, -1):
                candj = jcut0 | np.int32(1 << b)

                @plsc.parallel_loop(0, NV_ROW, unroll=4, carry=zeros16)
                def cj(i, c_c):
                    x = buf[pl.ds(i * 16, 16)]
                    m = (x == thr_f) & (lane + i * 16 >= candj)
                    return c_c + plsc.all_reduce_population_count(m)
                jcut0 = jnp.where(cj >= need, candj, jcut0)
            return jcut0

        tie_mode = ((n_eq != need).astype(jnp.int32))[0] != 0
        jcut = lax.cond(tie_mode, tie_search, lambda: zeros16)

        @pl.when(jnp.logical_not(tie_mode))
        def _mask_fast():
            @plsc.parallel_loop(0, NV_ROW, unroll=8)
            def _mk(i):
                x = buf[pl.ds(i * 16, 16)]
                buf[pl.ds(i * 16, 16)] = jnp.where(x >= thr_f, x, 0.0)

        @pl.when(tie_mode)
        def _mask_tie():
            @plsc.parallel_loop(0, NV_ROW, unroll=4)
            def _mk(i):
                x = buf[pl.ds(i * 16, 16)]
                keep = (x > thr_f) | ((x == thr_f) & (lane + i * 16 >= jcut))
                buf[pl.ds(i * 16, 16)] = jnp.where(keep, x, 0.0)

    base = wid * ROWS_PER_WORKER

    def in_dma(buf, row, sem):
        return pltpu.make_async_copy(s_hbm.at[row], buf, sem)

    def out_dma(buf, row, sem):
        return pltpu.make_async_copy(buf, o_hbm.at[row], sem)

    in_dma(rowa_v, base, in0_sem).start()

    def _pair(p, _):
        r0 = base + 2 * p
        in_dma(rowa_v, r0, in0_sem).wait()

        @pl.when(p > 0)
        def _w():
            out_dma(rowb_v, r0 - 1, out1_sem).wait()
        in_dma(rowb_v, r0 + 1, in1_sem).start()
        compute_row(rowa_v, r0)
        out_dma(rowa_v, r0, out0_sem).start()
        in_dma(rowb_v, r0 + 1, in1_sem).wait()

        def _recycle_a():
            out_dma(rowa_v, r0, out0_sem).wait()

            @pl.when(p == 0)
            def _n():
                in_dma(rowa_v, r0 + 2, in0_sem).start()

        compute_row(rowb_v, r0 + 1, mid_cb=_recycle_a)
        out_dma(rowb_v, r0 + 1, out1_sem).start()
        return 0

    lax.fori_loop(0, ROWS_PER_WORKER // 2, _pair, 0)
    out_dma(rowb_v, base + 3, out1_sem).wait()


@jax.jit
def kernel(s):
    mesh = plsc.VectorSubcoreMesh(core_axis_name="c", subcore_axis_name="s",
                                  num_cores=2, num_subcores=16)
    return pl.kernel(
        _sc_body,
        out_type=jax.ShapeDtypeStruct((BATCH_C, NEURONS_C), jnp.float32),
        mesh=mesh,
        compiler_params=pltpu.CompilerParams(needs_layout_passes=False),
        scratch_types=[
            pltpu.VMEM((NEURONS_C,), jnp.float32),
            pltpu.VMEM((NEURONS_C,), jnp.float32),
            pltpu.VMEM((NV_ROW,), jnp.float32),
            pltpu.VMEM((256,), jnp.float32),
            pltpu.VMEM((NV_ROW + 16,), jnp.int32),
            pltpu.VMEM((NEURONS_C + 64,), jnp.float32),
            pltpu.SemaphoreType.DMA,
            pltpu.SemaphoreType.DMA,
            pltpu.SemaphoreType.DMA,
            pltpu.SemaphoreType.DMA,
        ],
    )(s)
